# baseline (speedup 1.0000x reference)
import jax
import jax.numpy as jnp
from jax import lax
from jax.experimental import pallas as pl
from jax.experimental.pallas import tpu as pltpu
from jax.experimental.pallas import tpu_sc as plsc

B, L, D = 8, 512, 256
K = 16
HD = D // 2
HT = L // 2
ZROW = L
NTV = HT // 16


def _body(inp, heads, scal, out,
          inp_v, heads_v, scal_v, depth_v, apd_v, cidx_v, mrg_v, pmax_v,
          ol0, ol1, or0, or1, lsem, gsem0, gsem1, wsem0, wsem1):
    wid = lax.axis_index("s") * 2 + lax.axis_index("c")
    b = wid // 4
    th = (wid % 4) // 2
    chf = wid % 2
    tbase = th * HT
    iota = lax.iota(jnp.int32, 16)

    lh = pltpu.async_copy(
        inp.at[pl.ds(b * L, L), pl.ds(chf * HD, HD)],
        inp_v.at[pl.ds(0, L)], lsem)
    pltpu.sync_copy(heads.at[pl.ds(b * L, L)], heads_v)
    pltpu.sync_copy(scal, scal_v)

    p_vec = plsc.load_gather(scal_v, [jnp.full((16,), b, jnp.int32)])
    len_vec = plsc.load_gather(scal_v, [jnp.full((16,), b + 8, jnp.int32)])

    @plsc.parallel_loop(0, L // 16, unroll=2)
    def depth_body(tv):
        ids = iota + tv * 16
        cur = ids
        d = jnp.zeros((16,), jnp.int32)
        for _k in range(1, K):
            nxt = plsc.load_gather(heads_v, [cur])
            d = d + jnp.where(nxt != cur, 1, 0)
            cur = nxt
        depth_v[pl.ds(tv * 16, 16)] = d

    cur = p_vec
    cp = jnp.where(iota == 0, cur, 0)
    for k in range(1, K):
        cur = plsc.load_gather(heads_v, [cur])
        cp = jnp.where(iota == k, cur, cp)
    depth_p_vec = plsc.load_gather(depth_v, [p_vec])
    dvals = plsc.load_gather(depth_v, [cp])

    @plsc.parallel_loop(0, L // 16, unroll=2)
    def apd_init(tv):
        apd_v[pl.ds(tv * 16, 16)] = jnp.full((16,), -1, jnp.int32)
    plsc.store_scatter(apd_v, [cp], dvals)

    @plsc.parallel_loop(0, HT // 16, unroll=2)
    def tok_idx_body(tv):
        ids = iota + tbase + tv * 16
        cur = ids
        lca = jnp.full((16,), -1, jnp.int32)
        for k in range(K):
            av = plsc.load_gather(apd_v, [cur])
            lca = jnp.maximum(lca, av)
            cidx_v[k, pl.ds(tv * 16, 16)] = cur
            if k < K - 1:
                cur = plsc.load_gather(heads_v, [cur])
        dmy = plsc.load_gather(depth_v, [ids])
        sl = dmy - lca
        sr = depth_p_vec - lca
        mr = jnp.clip(sr, 0, K - 1)
        valid = ids < len_vec
        for k in range(K):
            raw = cidx_v[k, pl.ds(tv * 16, 16)]
            g = raw if k == 0 else jnp.where(k <= sl, raw, ids)
            cidx_v[k, pl.ds(tv * 16, 16)] = jnp.where(valid, g, ZROW)
        mrg_v[pl.ds(tv * 16, 16)] = jnp.where(valid, mr, K)

    zf = jnp.zeros((16,), jnp.float32)
    for cc in range(HD // 16):
        inp_v[ZROW, pl.ds(cc * 16, 16)] = zf

    lh.wait()

    @plsc.parallel_loop(0, HD, unroll=2)
    def pmax_body(c):
        pmax_v[c, pl.ds(16, 16)] = zf
        col = plsc.load_gather(inp_v, [cp, jnp.full((16,), c, jnp.int32)])
        pmax_v[c, pl.ds(0, 16)] = plsc.cummax(col)

    obl = (ol0, ol1)
    obr = (or0, or1)
    wsems = (wsem0, wsem1)
    wh = [None] * NTV
    for tv in range(NTV):
        q = tv % 2
        if tv >= 2:
            wh[tv - 2][0].wait()
            wh[tv - 2][1].wait()
        idx = [cidx_v[k, pl.ds(tv * 16, 16)] for k in range(K)]
        mrg = mrg_v[pl.ds(tv * 16, 16)]
        olv = obl[q]
        orv = obr[q]

        @plsc.parallel_loop(0, HD, unroll=4)
        def c_body(c, idx=idx, mrg=mrg, olv=olv, orv=orv):
            cvec = jnp.full((16,), c, jnp.int32)
            g = [plsc.load_gather(inp_v, [idx[k], cvec]) for k in range(K)]
            while len(g) > 1:
                g = [jnp.maximum(g[i], g[i + 1]) for i in range(0, len(g), 2)]
            plsc.store_scatter(olv, [iota, cvec], g[0])
            rv = plsc.load_gather(pmax_v, [cvec, mrg])
            plsc.store_scatter(orv, [iota, cvec], rv)

        gt0 = b * L + tbase + tv * 16
        wh[tv] = (
            pltpu.async_copy(
                olv, out.at[pl.ds(gt0, 16), pl.ds(chf * HD, HD)], wsems[q]),
            pltpu.async_copy(
                orv, out.at[pl.ds(gt0, 16), pl.ds(D + chf * HD, HD)], wsems[q]),
        )
    wh[NTV - 2][0].wait()
    wh[NTV - 2][1].wait()
    wh[NTV - 1][0].wait()
    wh[NTV - 1][1].wait()


_call = pl.kernel(
    _body,
    out_type=jax.ShapeDtypeStruct((B * L, 2 * D), jnp.float32),
    mesh=plsc.VectorSubcoreMesh(core_axis_name="c", subcore_axis_name="s"),
    compiler_params=pltpu.CompilerParams(needs_layout_passes=False),
    scratch_types=[
        pltpu.VMEM((L + 8, HD), jnp.float32),
        pltpu.VMEM((L,), jnp.int32),
        pltpu.VMEM((16,), jnp.int32),
        pltpu.VMEM((L,), jnp.int32),
        pltpu.VMEM((L,), jnp.int32),
        pltpu.VMEM((K, HT), jnp.int32),
        pltpu.VMEM((HT,), jnp.int32),
        pltpu.VMEM((HD, 32), jnp.float32),
        pltpu.VMEM((16, HD), jnp.float32),
        pltpu.VMEM((16, HD), jnp.float32),
        pltpu.VMEM((16, HD), jnp.float32),
        pltpu.VMEM((16, HD), jnp.float32),
        pltpu.SemaphoreType.DMA,
        pltpu.SemaphoreType.DMA,
        pltpu.SemaphoreType.DMA,
        pltpu.SemaphoreType.DMA,
        pltpu.SemaphoreType.DMA,
    ],
)


def kernel(inputs, heads, predicates, lengths):
    inp = inputs.reshape(B * L, D)
    heads_f = heads.reshape(B * L).astype(jnp.int32)
    scal = jnp.concatenate(
        [predicates.astype(jnp.int32), lengths.astype(jnp.int32)])
    out = _call(inp, heads_f, scal)
    return out.reshape(B, L, 2 * D)


# speedup vs baseline: 2.7495x; 2.7495x over previous
import jax
import jax.numpy as jnp
from jax import lax
from jax.experimental import pallas as pl
from jax.experimental.pallas import tpu as pltpu
from jax.experimental.pallas import tpu_sc as plsc

B, L, D = 8, 512, 256
K = 16
HD = D // 2
HT = L // 2
ZROW = L
NTV = HT // 16
NC = HD // 16


def _body(inp, heads, scal, out,
          inp_v, heads_v, scal_v, depth_v, apd_v, cidx_v, mrg_v, pmax_v,
          obuf, lsem, wsem):
    wid = lax.axis_index("s") * 2 + lax.axis_index("c")
    b = wid // 4
    th = (wid % 4) // 2
    chf = wid % 2
    tbase = th * HT
    gbase = b * L + tbase
    iota = lax.iota(jnp.int32, 16)

    lh = pltpu.async_copy(
        inp.at[pl.ds(b * L, L), pl.ds(chf * HD, HD)],
        inp_v.at[pl.ds(0, L)], lsem)
    pltpu.sync_copy(heads.at[pl.ds(b * L, L)], heads_v)
    pltpu.sync_copy(scal, scal_v)

    p_vec = plsc.load_gather(scal_v, [jnp.full((16,), b, jnp.int32)])
    len_vec = plsc.load_gather(scal_v, [jnp.full((16,), b + 8, jnp.int32)])

    @plsc.parallel_loop(0, L // 16, unroll=2)
    def depth_body(tv):
        ids = iota + tv * 16
        cur = ids
        d = jnp.zeros((16,), jnp.int32)
        for _k in range(1, K):
            nxt = plsc.load_gather(heads_v, [cur])
            d = d + jnp.where(nxt != cur, 1, 0)
            cur = nxt
        depth_v[pl.ds(tv * 16, 16)] = d

    cur = p_vec
    cp = jnp.where(iota == 0, cur, 0)
    for k in range(1, K):
        cur = plsc.load_gather(heads_v, [cur])
        cp = jnp.where(iota == k, cur, cp)
    depth_p_vec = plsc.load_gather(depth_v, [p_vec])
    dvals = plsc.load_gather(depth_v, [cp])

    @plsc.parallel_loop(0, L // 16, unroll=2)
    def apd_init(tv):
        apd_v[pl.ds(tv * 16, 16)] = jnp.full((16,), -1, jnp.int32)
    plsc.store_scatter(apd_v, [cp], dvals)

    @plsc.parallel_loop(0, NTV, unroll=2)
    def tok_idx_body(tv):
        ids = iota + tbase + tv * 16
        cur = ids
        lca = jnp.full((16,), -1, jnp.int32)
        raws = []
        for k in range(K):
            av = plsc.load_gather(apd_v, [cur])
            lca = jnp.maximum(lca, av)
            raws.append(cur)
            if k < K - 1:
                cur = plsc.load_gather(heads_v, [cur])
        dmy = plsc.load_gather(depth_v, [ids])
        sl = dmy - lca
        sr = depth_p_vec - lca
        mr = jnp.clip(sr, 0, K - 1)
        valid = ids < len_vec
        for k in range(K):
            g = raws[k] if k == 0 else jnp.where(k <= sl, raws[k], ids)
            cidx_v[k, pl.ds(tv * 16, 16)] = jnp.where(valid, g, ZROW)
        mrg_v[pl.ds(tv * 16, 16)] = jnp.where(valid, mr, K)

    zf = jnp.zeros((16,), jnp.float32)
    for cc in range(NC):
        inp_v[ZROW, pl.ds(cc * 16, 16)] = zf

    lh.wait()

    for cc in range(NC):
        pmax_v[K, pl.ds(cc * 16, 16)] = zf
    for m in range(K):
        r = cp[m]
        for cc in range(NC):
            row = inp_v[r, pl.ds(cc * 16, 16)]
            if m > 0:
                row = jnp.maximum(row, pmax_v[m - 1, pl.ds(cc * 16, 16)])
            pmax_v[m, pl.ds(cc * 16, 16)] = row

    @plsc.parallel_loop(0, NTV)
    def left_body(tv):
        idxv = [cidx_v[k, pl.ds(tv * 16, 16)] for k in range(K)]
        for j in range(16):
            rs = [idxv[k][j] for k in range(K)]
            row = tv * 16 + j
            for cc in range(NC):
                g = [inp_v[rs[k], pl.ds(cc * 16, 16)] for k in range(K)]
                while len(g) > 1:
                    g = [jnp.maximum(g[i], g[i + 1])
                         for i in range(0, len(g), 2)]
                obuf[row, pl.ds(cc * 16, 16)] = g[0]

    lwh = pltpu.async_copy(
        obuf, out.at[pl.ds(gbase, HT), pl.ds(chf * HD, HD)], wsem)
    lwh.wait()

    @plsc.parallel_loop(0, NTV)
    def right_body(tv):
        mrgv = mrg_v[pl.ds(tv * 16, 16)]
        for j in range(16):
            m = mrgv[j]
            row = tv * 16 + j
            for cc in range(NC):
                obuf[row, pl.ds(cc * 16, 16)] = pmax_v[m, pl.ds(cc * 16, 16)]

    pltpu.sync_copy(
        obuf, out.at[pl.ds(gbase, HT), pl.ds(D + chf * HD, HD)])


_call = pl.kernel(
    _body,
    out_type=jax.ShapeDtypeStruct((B * L, 2 * D), jnp.float32),
    mesh=plsc.VectorSubcoreMesh(core_axis_name="c", subcore_axis_name="s"),
    compiler_params=pltpu.CompilerParams(needs_layout_passes=False),
    scratch_types=[
        pltpu.VMEM((L + 8, HD), jnp.float32),
        pltpu.VMEM((L,), jnp.int32),
        pltpu.VMEM((16,), jnp.int32),
        pltpu.VMEM((L,), jnp.int32),
        pltpu.VMEM((L,), jnp.int32),
        pltpu.VMEM((K, HT), jnp.int32),
        pltpu.VMEM((HT,), jnp.int32),
        pltpu.VMEM((K + 1, HD), jnp.float32),
        pltpu.VMEM((HT, HD), jnp.float32),
        pltpu.SemaphoreType.DMA,
        pltpu.SemaphoreType.DMA,
    ],
)


def kernel(inputs, heads, predicates, lengths):
    inp = inputs.reshape(B * L, D)
    heads_f = heads.reshape(B * L).astype(jnp.int32)
    scal = jnp.concatenate(
        [predicates.astype(jnp.int32), lengths.astype(jnp.int32)])
    out = _call(inp, heads_f, scal)
    return out.reshape(B, L, 2 * D)
